# baseline (speedup 1.0000x reference)
import functools

import jax
import jax.numpy as jnp
from jax import lax
from jax.experimental import pallas as pl
from jax.experimental.pallas import tpu as pltpu
from jax.experimental.pallas import tpu_sc as plsc

ROWS = 64
ROW_LEN = 32768
OUT_K = 256
LANES = 16
NV = OUT_K // LANES
NCHUNK = ROW_LEN // OUT_K
NWORKERS = 32
ROWS_PER_W = ROWS // NWORKERS


def _vsort(v, desc):
    k, _ = plsc.sort_key_val(v, v, descending=desc)
    return k


def _bitonic_merge(s, desc):
    m = len(s)
    if m == 1:
        return [_vsort(s[0], desc)]
    h = m // 2
    if desc:
        a = [jnp.maximum(s[i], s[i + h]) for i in range(h)]
        b = [jnp.minimum(s[i], s[i + h]) for i in range(h)]
    else:
        a = [jnp.minimum(s[i], s[i + h]) for i in range(h)]
        b = [jnp.maximum(s[i], s[i + h]) for i in range(h)]
    return _bitonic_merge(a, desc) + _bitonic_merge(b, desc)


def _build_sorted(vs, desc):
    if len(vs) == 1:
        return [_vsort(vs[0], desc)]
    h = len(vs) // 2
    a = _build_sorted(vs[:h], True)
    b = _build_sorted(vs[h:], False)
    return _bitonic_merge(a + b, desc)


def _topk_merge(acc_desc, chunk_asc):
    hi = [jnp.maximum(x, y) for x, y in zip(acc_desc, chunk_asc)]
    return _bitonic_merge(hi, True)


def _make_kernel():
    mesh = plsc.VectorSubcoreMesh(
        core_axis_name="c", subcore_axis_name="s", num_cores=1)

    @functools.partial(
        pl.kernel,
        mesh=mesh,
        out_type=jax.ShapeDtypeStruct((ROWS, OUT_K), jnp.float32),
        scratch_types=[
            pltpu.VMEM((ROW_LEN,), jnp.float32),
            pltpu.VMEM((ROW_LEN + OUT_K,), jnp.float32),
            pltpu.VMEM((OUT_K,), jnp.float32),
        ],
        compiler_params=pltpu.CompilerParams(needs_layout_passes=False),
    )
    def topk_rows(x_hbm, out_hbm, row_v, cand_v, out_v):
        sid = lax.axis_index("s")
        iota = lax.iota(jnp.int32, LANES)
        ninf = jnp.full((LANES,), -jnp.inf, jnp.float32)

        def do_row(t, _):
            row = sid * (ROWS // 16) + t
            pltpu.sync_copy(x_hbm.at[row], row_v)

            def p1(b, m):
                return tuple(
                    jnp.maximum(m[i], row_v[pl.ds(b * OUT_K + i * LANES, LANES)])
                    for i in range(NV))

            m = lax.fori_loop(
                1, NCHUNK, p1,
                tuple(row_v[pl.ds(i * LANES, LANES)] for i in range(NV)))
            m = list(m)
            while len(m) > 1:
                h = len(m) // 2
                m = [jnp.minimum(m[i], m[i + h]) for i in range(h)]
            thr = jnp.min(m[0])

            def p2(b, off):
                vals, masks, cums, pops = [], [], [], []
                for i in range(NV):
                    v = row_v[pl.ds(b * OUT_K + i * LANES, LANES)]
                    mask = v >= thr
                    vals.append(v)
                    masks.append(mask)
                    cums.append(plsc.cumsum(mask.astype(jnp.int32)))
                    pops.append(plsc.all_reduce_population_count(mask))
                pref = off
                for i in range(NV):
                    idx = pref + cums[i] - 1
                    plsc.store_scatter(cand_v, [idx], vals[i], mask=masks[i])
                    pref = pref + pops[i]
                tot = pops
                while len(tot) > 1:
                    tot = [tot[i] + tot[i + len(tot) // 2]
                           for i in range(len(tot) // 2)]
                return off + tot[0]

            off = lax.fori_loop(0, NCHUNK, p2, jnp.zeros((LANES,), jnp.int32))

            for i in range(NV):
                plsc.store_scatter(cand_v, [off + (iota + i * LANES)], ninf)

            count = jnp.max(off)
            nch = (count + (OUT_K - 1)) // OUT_K

            def fold(j, acc):
                cur = _build_sorted(
                    [cand_v[pl.ds(j * OUT_K + i * LANES, LANES)]
                     for i in range(NV)], False)
                return tuple(_topk_merge(list(acc), cur))

            acc = lax.fori_loop(0, nch, fold, tuple(ninf for _ in range(NV)))

            for i in range(NV):
                out_v[pl.ds(i * LANES, LANES)] = acc[i]
            pltpu.sync_copy(out_v, out_hbm.at[row])
            return 0

        lax.fori_loop(0, ROWS // 16, do_row, 0)

    return topk_rows


_topk_kernel = _make_kernel()


def kernel(x):
    return _topk_kernel(x)


# speedup vs baseline: 1.3408x; 1.3408x over previous
import functools

import jax
import jax.numpy as jnp
from jax import lax
from jax.experimental import pallas as pl
from jax.experimental.pallas import tpu as pltpu
from jax.experimental.pallas import tpu_sc as plsc

ROWS = 64
ROW_LEN = 32768
OUT_K = 256
LANES = 16
NV = OUT_K // LANES
NCHUNK = ROW_LEN // OUT_K
NWORKERS = 32
ROWS_PER_W = ROWS // NWORKERS


def _vsort(v, desc):
    k, _ = plsc.sort_key_val(v, v, descending=desc)
    return k


def _bitonic_merge(s, desc):
    m = len(s)
    if m == 1:
        return [_vsort(s[0], desc)]
    h = m // 2
    if desc:
        a = [jnp.maximum(s[i], s[i + h]) for i in range(h)]
        b = [jnp.minimum(s[i], s[i + h]) for i in range(h)]
    else:
        a = [jnp.minimum(s[i], s[i + h]) for i in range(h)]
        b = [jnp.maximum(s[i], s[i + h]) for i in range(h)]
    return _bitonic_merge(a, desc) + _bitonic_merge(b, desc)


def _build_sorted(vs, desc):
    if len(vs) == 1:
        return [_vsort(vs[0], desc)]
    h = len(vs) // 2
    a = _build_sorted(vs[:h], True)
    b = _build_sorted(vs[h:], False)
    return _bitonic_merge(a + b, desc)


def _topk_merge(acc_desc, chunk_asc):
    hi = [jnp.maximum(x, y) for x, y in zip(acc_desc, chunk_asc)]
    return _bitonic_merge(hi, True)


def _make_kernel():
    mesh = plsc.VectorSubcoreMesh(core_axis_name="c", subcore_axis_name="s")

    @functools.partial(
        pl.kernel,
        mesh=mesh,
        out_type=jax.ShapeDtypeStruct((ROWS, OUT_K), jnp.float32),
        scratch_types=[
            pltpu.VMEM((ROW_LEN,), jnp.float32),
            pltpu.VMEM((ROW_LEN,), jnp.float32),
            pltpu.VMEM((ROW_LEN + OUT_K,), jnp.float32),
            pltpu.VMEM((OUT_K,), jnp.float32),
            pltpu.VMEM((OUT_K,), jnp.float32),
            pltpu.SemaphoreType.DMA,
            pltpu.SemaphoreType.DMA,
            pltpu.SemaphoreType.DMA,
            pltpu.SemaphoreType.DMA,
        ],
        compiler_params=pltpu.CompilerParams(needs_layout_passes=False),
    )
    def topk_rows(x_hbm, out_hbm, row0_v, row1_v, cand_v, out0_v, out1_v,
                  in_sem0, in_sem1, out_sem0, out_sem1):
        wid = lax.axis_index("s") * 2 + lax.axis_index("c")
        iota = lax.iota(jnp.int32, LANES)
        ninf = jnp.full((LANES,), -jnp.inf, jnp.float32)

        row0 = wid * ROWS_PER_W
        in0 = pltpu.async_copy(x_hbm.at[row0], row0_v, in_sem0)
        in1 = pltpu.async_copy(x_hbm.at[row0 + 1], row1_v, in_sem1)

        def do_row(row_v, out_v, in_cp, out_sem, row):
            in_cp.wait()

            def p1(b, m):
                return tuple(
                    jnp.maximum(m[i], row_v[pl.ds(b * OUT_K + i * LANES, LANES)])
                    for i in range(NV))

            m = lax.fori_loop(
                1, NCHUNK, p1,
                tuple(row_v[pl.ds(i * LANES, LANES)] for i in range(NV)))
            m = list(m)
            while len(m) > 1:
                h = len(m) // 2
                m = [jnp.minimum(m[i], m[i + h]) for i in range(h)]
            thr = jnp.min(m[0])

            def p2(b, off):
                vals, masks, cums, pops = [], [], [], []
                for i in range(NV):
                    v = row_v[pl.ds(b * OUT_K + i * LANES, LANES)]
                    mask = v >= thr
                    vals.append(v)
                    masks.append(mask)
                    cums.append(plsc.cumsum(mask.astype(jnp.int32)))
                    pops.append(plsc.all_reduce_population_count(mask))
                pref = off
                for i in range(NV):
                    idx = pref + cums[i] - 1
                    plsc.store_scatter(cand_v, [idx], vals[i], mask=masks[i])
                    pref = pref + pops[i]
                tot = pops
                while len(tot) > 1:
                    tot = [tot[i] + tot[i + len(tot) // 2]
                           for i in range(len(tot) // 2)]
                return off + tot[0]

            off = lax.fori_loop(0, NCHUNK, p2, jnp.zeros((LANES,), jnp.int32))

            for i in range(NV):
                plsc.store_scatter(cand_v, [off + (iota + i * LANES)], ninf)

            count = jnp.max(off)
            nch = (count + (OUT_K - 1)) // OUT_K

            def fold(j, acc):
                cur = _build_sorted(
                    [cand_v[pl.ds(j * OUT_K + i * LANES, LANES)]
                     for i in range(NV)], False)
                return tuple(_topk_merge(list(acc), cur))

            acc = lax.fori_loop(0, nch, fold, tuple(ninf for _ in range(NV)))

            for i in range(NV):
                out_v[pl.ds(i * LANES, LANES)] = acc[i]
            return pltpu.async_copy(out_v, out_hbm.at[row], out_sem)

        out0 = do_row(row0_v, out0_v, in0, out_sem0, row0)
        out1 = do_row(row1_v, out1_v, in1, out_sem1, row0 + 1)
        out0.wait()
        out1.wait()

    return topk_rows


_topk_kernel = _make_kernel()


def kernel(x):
    return _topk_kernel(x)


# speedup vs baseline: 1.4029x; 1.0463x over previous
import functools

import jax
import jax.numpy as jnp
from jax import lax
from jax.experimental import pallas as pl
from jax.experimental.pallas import tpu as pltpu
from jax.experimental.pallas import tpu_sc as plsc

ROWS = 64
ROW_LEN = 32768
OUT_K = 256
LANES = 16
NV = OUT_K // LANES
NCHUNK = ROW_LEN // OUT_K
NWORKERS = 32
ROWS_PER_W = ROWS // NWORKERS


def _vsort(v, desc):
    k, _ = plsc.sort_key_val(v, v, descending=desc)
    return k


def _bitonic_merge(s, desc):
    m = len(s)
    if m == 1:
        return [_vsort(s[0], desc)]
    h = m // 2
    if desc:
        a = [jnp.maximum(s[i], s[i + h]) for i in range(h)]
        b = [jnp.minimum(s[i], s[i + h]) for i in range(h)]
    else:
        a = [jnp.minimum(s[i], s[i + h]) for i in range(h)]
        b = [jnp.maximum(s[i], s[i + h]) for i in range(h)]
    return _bitonic_merge(a, desc) + _bitonic_merge(b, desc)


def _build_sorted(vs, desc):
    if len(vs) == 1:
        return [_vsort(vs[0], desc)]
    h = len(vs) // 2
    a = _build_sorted(vs[:h], True)
    b = _build_sorted(vs[h:], False)
    return _bitonic_merge(a + b, desc)


def _topk_merge(acc_desc, chunk_asc):
    hi = [jnp.maximum(x, y) for x, y in zip(acc_desc, chunk_asc)]
    return _bitonic_merge(hi, True)


def _make_kernel():
    mesh = plsc.VectorSubcoreMesh(core_axis_name="c", subcore_axis_name="s")

    @functools.partial(
        pl.kernel,
        mesh=mesh,
        out_type=jax.ShapeDtypeStruct((ROWS, OUT_K), jnp.float32),
        scratch_types=[
            pltpu.VMEM((ROW_LEN,), jnp.float32),
            pltpu.VMEM((ROW_LEN,), jnp.float32),
            pltpu.VMEM((ROW_LEN,), jnp.float32),
            pltpu.VMEM((OUT_K,), jnp.float32),
            pltpu.VMEM((OUT_K,), jnp.float32),
            pltpu.SemaphoreType.DMA,
            pltpu.SemaphoreType.DMA,
            pltpu.SemaphoreType.DMA,
            pltpu.SemaphoreType.DMA,
        ],
        compiler_params=pltpu.CompilerParams(needs_layout_passes=False),
    )
    def topk_rows(x_hbm, out_hbm, row0_v, row1_v, cand_v, out0_v, out1_v,
                  in_sem0, in_sem1, out_sem0, out_sem1):
        wid = lax.axis_index("s") * 2 + lax.axis_index("c")
        iota = lax.iota(jnp.int32, LANES)
        ninf = jnp.full((LANES,), -jnp.inf, jnp.float32)

        row0 = wid * ROWS_PER_W
        in0 = pltpu.async_copy(x_hbm.at[row0], row0_v, in_sem0)
        in1 = pltpu.async_copy(x_hbm.at[row0 + 1], row1_v, in_sem1)

        def do_row(row_v, out_v, in_cp, out_sem, row):
            in_cp.wait()

            def p1(b, m):
                return tuple(
                    jnp.maximum(m[i], row_v[pl.ds(b * OUT_K + i * LANES, LANES)])
                    for i in range(NV))

            m = lax.fori_loop(
                1, NCHUNK, p1,
                tuple(row_v[pl.ds(i * LANES, LANES)] for i in range(NV)))
            m = list(m)
            while len(m) > 1:
                h = len(m) // 2
                m = [jnp.minimum(m[i], m[i + h]) for i in range(h)]
            thr = jnp.min(m[0])

            def p2(b, ptr16):
                items = []
                for i in range(NV):
                    v = row_v[pl.ds(b * OUT_K + i * LANES, LANES)]
                    mask = v >= thr
                    m16 = jnp.where(mask, jnp.int32(16), jnp.int32(0))
                    items.append((v, mask, m16))
                pref = ptr16
                for v, mask, m16 in items:
                    plsc.store_scatter(cand_v, [pref], v, mask=mask)
                    pref = pref + m16
                tot = [m16 for _, _, m16 in items]
                while len(tot) > 1:
                    tot = [tot[i] + tot[i + len(tot) // 2]
                           for i in range(len(tot) // 2)]
                return ptr16 + tot[0]

            ptr16 = lax.fori_loop(0, NCHUNK, p2, iota)

            depth = lax.shift_right_logical(ptr16 - iota, 4)
            dmax = jnp.max(depth)
            nch = (dmax + (LANES - 1)) // LANES
            dtarget16 = nch * LANES * LANES
            steps = nch * LANES - jnp.min(depth)

            def padb(k, p16):
                mask = p16 < dtarget16
                plsc.store_scatter(cand_v, [p16], ninf, mask=mask)
                return p16 + jnp.where(mask, jnp.int32(16), jnp.int32(0))

            lax.fori_loop(0, steps, padb, ptr16)

            def fold(j, acc):
                cur = _build_sorted(
                    [cand_v[pl.ds(j * OUT_K + i * LANES, LANES)]
                     for i in range(NV)], False)
                return tuple(_topk_merge(list(acc), cur))

            acc = lax.fori_loop(0, nch, fold, tuple(ninf for _ in range(NV)))

            for i in range(NV):
                out_v[pl.ds(i * LANES, LANES)] = acc[i]
            return pltpu.async_copy(out_v, out_hbm.at[row], out_sem)

        out0 = do_row(row0_v, out0_v, in0, out_sem0, row0)
        out1 = do_row(row1_v, out1_v, in1, out_sem1, row0 + 1)
        out0.wait()
        out1.wait()

    return topk_rows


_topk_kernel = _make_kernel()


def kernel(x):
    return _topk_kernel(x)
